# baseline (speedup 1.0000x reference)
import jax
import jax.numpy as jnp
from jax import lax
from jax.experimental import pallas as pl
from jax.experimental.pallas import tpu as pltpu
from jax.experimental.pallas import tpu_sc as plsc

E_CHAR = 50
EMBED = 256
CHAR_VOCAB = 96
KSIZE = 5
L = 21
T_OUT = L - KSIZE + 1

NW = 32
N_SC = 10240
WPT = N_SC // NW
GRP = 16
NGRP = WPT // GRP
EPAIRS = EMBED // 2
EP_CHUNK = 2
NCHUNK = EPAIRS // EP_CHUNK
TROW = KSIZE * EPAIRS + 1


def _table_body(emb_ref, w2_ref, cb_ref, out_ref):
    t = jnp.dot(emb_ref[...], w2_ref[...], preferred_element_type=jnp.float32)
    out_ref[...] = (t + cb_ref[...][None, :]).astype(jnp.bfloat16)


def _sc_body(ids_hbm, table_hbm, out_hbm, table_v, ids_v, vbs_v, out_v,
             sem0, sem1):
    wid = lax.axis_index("s") * 2 + lax.axis_index("c")
    pltpu.sync_copy(table_hbm, table_v)
    pltpu.sync_copy(ids_hbm.at[wid], ids_v)
    lanes = lax.iota(jnp.int32, 16)
    sems = (sem0, sem1)

    def do_group(g, buf, sem, first):
        for j in range(L):
            vbs_v[j] = ids_v[j, pl.ds(g * GRP, GRP)]
        dst = out_hbm.at[pl.ds(wid * WPT + g * GRP, GRP)]

        def chunk_body(c, carry2):
            ep0 = c * EP_CHUNK
            m = [jnp.zeros((2 * GRP,), jnp.bfloat16)] * EP_CHUNK
            for t in range(T_OUT):
                acc = [None] * EP_CHUNK
                for k in range(KSIZE):
                    base = vbs_v[t + k] + (k * EPAIRS + ep0)
                    for j in range(EP_CHUNK):
                        v = plsc.load_gather(table_v, [base + j])
                        v = plsc.bitcast(v, jnp.bfloat16)
                        acc[j] = v if k == 0 else acc[j] + v
                for j in range(EP_CHUNK):
                    m[j] = jnp.maximum(m[j], acc[j])
            for j in range(EP_CHUNK):
                lo, hi = plsc.unpack(m[j], format=plsc.PackFormat.INTERLEAVED)
                col = 2 * (ep0 + j)
                plsc.store_scatter(out_v.at[buf],
                                   [lanes, jnp.full((GRP,), 0, jnp.int32) + col], lo)
                plsc.store_scatter(out_v.at[buf],
                                   [lanes, jnp.full((GRP,), 1, jnp.int32) + col], hi)
            return carry2

        @pl.when(jnp.logical_not(first))
        def _():
            pltpu.make_async_copy(out_v.at[buf], dst, sem).wait()
        lax.fori_loop(0, NCHUNK, chunk_body, 0, unroll=False)
        pltpu.async_copy(out_v.at[buf], dst, sem)

    def group_pair(i, carry):
        do_group(2 * i, 0, sems[0], i == 0)
        do_group(2 * i + 1, 1, sems[1], i == 0)
        return carry

    lax.fori_loop(0, NGRP // 2, group_pair, 0, unroll=False)
    last = out_hbm.at[pl.ds(wid * WPT + (NGRP - 2) * GRP, GRP)]
    pltpu.make_async_copy(out_v.at[0], last, sems[0]).wait()
    pltpu.make_async_copy(out_v.at[1], last, sems[1]).wait()


def _conv_block_body(ids_ref, emb_ref, convw_ref, convb_ref,
                     wp_ref, bp_ref, wg_ref, bg_ref, out_ref):
    nb = ids_ref.shape[0]
    ids = ids_ref[...]
    iota_v = jax.lax.broadcasted_iota(jnp.int32, (nb, L, CHAR_VOCAB), 2)
    oh = (ids[:, :, None] == iota_v).astype(jnp.bfloat16)
    e = jnp.dot(oh.reshape(nb * L, CHAR_VOCAB),
                emb_ref[...].astype(jnp.bfloat16),
                preferred_element_type=jnp.float32)
    e = e.astype(jnp.bfloat16).reshape(nb, L, E_CHAR)
    acc = jnp.zeros((nb * T_OUT, EMBED), jnp.float32)
    for k in range(KSIZE):
        ek = e[:, k:k + T_OUT, :].reshape(nb * T_OUT, E_CHAR)
        acc = acc + jnp.dot(ek, convw_ref[k].astype(jnp.bfloat16),
                            preferred_element_type=jnp.float32)
    acc = acc + convb_ref[...][None, :]
    acc = jnp.maximum(acc, 0.0).reshape(nb, T_OUT, EMBED)
    xc = jnp.max(acc, axis=1)
    proj = jnp.maximum(
        jnp.dot(xc, wp_ref[...], preferred_element_type=jnp.float32)
        + bp_ref[...][None, :], 0.0)
    gate = jax.nn.sigmoid(
        jnp.dot(xc, wg_ref[...], preferred_element_type=jnp.float32)
        + bg_ref[...][None, :])
    out_ref[...] = gate * proj + (1.0 - gate) * xc


def _highway_body(xc_ref, wp_ref, bp_ref, wg_ref, bg_ref, out_ref):
    xc = xc_ref[...]
    proj = jnp.maximum(
        jnp.dot(xc, wp_ref[...], preferred_element_type=jnp.float32)
        + bp_ref[...][None, :], 0.0)
    gate = jax.nn.sigmoid(
        jnp.dot(xc, wg_ref[...], preferred_element_type=jnp.float32)
        + bg_ref[...][None, :])
    out_ref[...] = gate * proj + (1.0 - gate) * xc


def kernel(input, emb_table, conv_w, conv_b, w_proj, b_proj, w_gate, b_gate):
    s, b, l = input.shape
    n = s * b

    w2 = jnp.transpose(conv_w, (1, 2, 0)).reshape(E_CHAR, KSIZE * EMBED)
    cb = jnp.tile(conv_b / KSIZE, KSIZE)
    t_bf = pl.pallas_call(
        _table_body,
        out_shape=jax.ShapeDtypeStruct((CHAR_VOCAB, KSIZE * EMBED), jnp.bfloat16),
    )(emb_table, w2, cb)
    t_packed = lax.bitcast_convert_type(
        t_bf.reshape(CHAR_VOCAB, KSIZE * EPAIRS, 2), jnp.float32)
    t_packed = jnp.pad(t_packed, ((0, 0), (0, TROW - KSIZE * EPAIRS))
                       ).reshape(CHAR_VOCAB * TROW)

    all_ids = input.reshape(n, l).astype(jnp.int32)
    ids = jnp.swapaxes(all_ids[:N_SC].reshape(NW, WPT, L), 1, 2) * TROW
    mesh = plsc.VectorSubcoreMesh(core_axis_name="c", subcore_axis_name="s",
                                  num_cores=2, num_subcores=16)
    xc_sc = pl.kernel(
        _sc_body,
        out_type=jax.ShapeDtypeStruct((N_SC, EMBED), jnp.float32),
        mesh=mesh,
        compiler_params=pltpu.CompilerParams(needs_layout_passes=False),
        scratch_types=[
            pltpu.VMEM((CHAR_VOCAB * TROW,), jnp.float32),
            pltpu.VMEM((L, WPT), jnp.int32),
            pltpu.VMEM((L, GRP), jnp.int32),
            pltpu.VMEM((2, GRP, EMBED), jnp.float32),
            pltpu.SemaphoreType.DMA,
            pltpu.SemaphoreType.DMA,
        ],
    )(ids, t_packed)

    n_tc = n - N_SC
    cblk = 512
    convw_t = jnp.transpose(conv_w, (2, 1, 0))
    xc_tc = pl.pallas_call(
        _conv_block_body,
        grid=(n_tc // cblk,),
        in_specs=[
            pl.BlockSpec((cblk, l), lambda i: (i, 0)),
            pl.BlockSpec((CHAR_VOCAB, E_CHAR), lambda i: (0, 0)),
            pl.BlockSpec((KSIZE, E_CHAR, EMBED), lambda i: (0, 0, 0)),
            pl.BlockSpec((EMBED,), lambda i: (0,)),
            pl.BlockSpec((EMBED, EMBED), lambda i: (0, 0)),
            pl.BlockSpec((EMBED,), lambda i: (0,)),
            pl.BlockSpec((EMBED, EMBED), lambda i: (0, 0)),
            pl.BlockSpec((EMBED,), lambda i: (0,)),
        ],
        out_specs=pl.BlockSpec((cblk, EMBED), lambda i: (i, 0)),
        out_shape=jax.ShapeDtypeStruct((n_tc, EMBED), jnp.float32),
    )(all_ids[N_SC:], emb_table, convw_t, conv_b,
      w_proj.T, b_proj, w_gate.T, b_gate)

    blk = 2048
    out_sc = pl.pallas_call(
        _highway_body,
        grid=(N_SC // blk,),
        in_specs=[
            pl.BlockSpec((blk, EMBED), lambda i: (i, 0)),
            pl.BlockSpec((EMBED, EMBED), lambda i: (0, 0)),
            pl.BlockSpec((EMBED,), lambda i: (0,)),
            pl.BlockSpec((EMBED, EMBED), lambda i: (0, 0)),
            pl.BlockSpec((EMBED,), lambda i: (0,)),
        ],
        out_specs=pl.BlockSpec((blk, EMBED), lambda i: (i, 0)),
        out_shape=jax.ShapeDtypeStruct((N_SC, EMBED), jnp.float32),
    )(xc_sc, w_proj.T, b_proj, w_gate.T, b_gate)
    out = jnp.concatenate([out_sc, xc_tc], axis=0)
    return out.reshape(s, b, EMBED)


# speedup vs baseline: 1.0085x; 1.0085x over previous
import jax
import jax.numpy as jnp
from jax import lax
from jax.experimental import pallas as pl
from jax.experimental.pallas import tpu as pltpu
from jax.experimental.pallas import tpu_sc as plsc

E_CHAR = 50
EMBED = 256
CHAR_VOCAB = 96
KSIZE = 5
L = 21
T_OUT = L - KSIZE + 1

NW = 32
N_SC = 10240
WPT = N_SC // NW
GRP = 16
NGRP = WPT // GRP
EPAIRS = EMBED // 2
EP_CHUNK = 2
NCHUNK = EPAIRS // EP_CHUNK
TROW = KSIZE * EPAIRS + 1


def _table_body(emb_ref, w2_ref, cb_ref, out_ref):
    t = jnp.dot(emb_ref[...], w2_ref[...], preferred_element_type=jnp.float32)
    out_ref[...] = (t + cb_ref[...][None, :]).astype(jnp.bfloat16)


def _sc_body(ids_hbm, table_hbm, out_hbm, table_v, ids_v, out_v,
             sem0, sem1):
    wid = lax.axis_index("s") * 2 + lax.axis_index("c")
    pltpu.sync_copy(table_hbm, table_v)
    pltpu.sync_copy(ids_hbm.at[wid], ids_v)
    lanes = lax.iota(jnp.int32, 16)
    sems = (sem0, sem1)

    def do_group(g, buf, sem, first):
        vb = [ids_v[j, pl.ds(g * GRP, GRP)] for j in range(L)]
        dst = out_hbm.at[pl.ds(wid * WPT + g * GRP, GRP)]

        def chunk_body(c, carry2):
            ep0 = c * EP_CHUNK
            m = [jnp.zeros((2 * GRP,), jnp.bfloat16)] * EP_CHUNK
            for t in range(T_OUT):
                acc = [None] * EP_CHUNK
                for k in range(KSIZE):
                    base = vb[t + k] + (k * EPAIRS + ep0)
                    for j in range(EP_CHUNK):
                        v = plsc.load_gather(table_v, [base + j])
                        v = plsc.bitcast(v, jnp.bfloat16)
                        acc[j] = v if k == 0 else acc[j] + v
                for j in range(EP_CHUNK):
                    m[j] = jnp.maximum(m[j], acc[j])
            for j in range(EP_CHUNK):
                lo, hi = plsc.unpack(m[j], format=plsc.PackFormat.INTERLEAVED)
                col = 2 * (ep0 + j)
                plsc.store_scatter(out_v.at[buf],
                                   [lanes, jnp.full((GRP,), 0, jnp.int32) + col], lo)
                plsc.store_scatter(out_v.at[buf],
                                   [lanes, jnp.full((GRP,), 1, jnp.int32) + col], hi)
            return carry2

        @pl.when(jnp.logical_not(first))
        def _():
            pltpu.make_async_copy(out_v.at[buf], dst, sem).wait()
        lax.fori_loop(0, NCHUNK, chunk_body, 0, unroll=False)
        pltpu.async_copy(out_v.at[buf], dst, sem)

    def group_pair(i, carry):
        do_group(2 * i, 0, sems[0], i == 0)
        do_group(2 * i + 1, 1, sems[1], i == 0)
        return carry

    lax.fori_loop(0, NGRP // 2, group_pair, 0, unroll=False)
    last = out_hbm.at[pl.ds(wid * WPT + (NGRP - 2) * GRP, GRP)]
    pltpu.make_async_copy(out_v.at[0], last, sems[0]).wait()
    pltpu.make_async_copy(out_v.at[1], last, sems[1]).wait()


def _conv_block_body(ids_ref, emb_ref, convw_ref, convb_ref,
                     wp_ref, bp_ref, wg_ref, bg_ref, out_ref):
    nb = ids_ref.shape[0]
    ids = ids_ref[...]
    iota_v = jax.lax.broadcasted_iota(jnp.int32, (nb, L, CHAR_VOCAB), 2)
    oh = (ids[:, :, None] == iota_v).astype(jnp.bfloat16)
    e = jnp.dot(oh.reshape(nb * L, CHAR_VOCAB),
                emb_ref[...].astype(jnp.bfloat16),
                preferred_element_type=jnp.float32)
    e = e.astype(jnp.bfloat16).reshape(nb, L, E_CHAR)
    acc = jnp.zeros((nb * T_OUT, EMBED), jnp.float32)
    for k in range(KSIZE):
        ek = e[:, k:k + T_OUT, :].reshape(nb * T_OUT, E_CHAR)
        acc = acc + jnp.dot(ek, convw_ref[k].astype(jnp.bfloat16),
                            preferred_element_type=jnp.float32)
    acc = acc + convb_ref[...][None, :]
    acc = jnp.maximum(acc, 0.0).reshape(nb, T_OUT, EMBED)
    xc = jnp.max(acc, axis=1)
    proj = jnp.maximum(
        jnp.dot(xc, wp_ref[...], preferred_element_type=jnp.float32)
        + bp_ref[...][None, :], 0.0)
    gate = jax.nn.sigmoid(
        jnp.dot(xc, wg_ref[...], preferred_element_type=jnp.float32)
        + bg_ref[...][None, :])
    out_ref[...] = gate * proj + (1.0 - gate) * xc


def _highway_body(xc_ref, wp_ref, bp_ref, wg_ref, bg_ref, out_ref):
    xc = xc_ref[...]
    proj = jnp.maximum(
        jnp.dot(xc, wp_ref[...], preferred_element_type=jnp.float32)
        + bp_ref[...][None, :], 0.0)
    gate = jax.nn.sigmoid(
        jnp.dot(xc, wg_ref[...], preferred_element_type=jnp.float32)
        + bg_ref[...][None, :])
    out_ref[...] = gate * proj + (1.0 - gate) * xc


def kernel(input, emb_table, conv_w, conv_b, w_proj, b_proj, w_gate, b_gate):
    s, b, l = input.shape
    n = s * b

    w2 = jnp.transpose(conv_w, (1, 2, 0)).reshape(E_CHAR, KSIZE * EMBED)
    cb = jnp.tile(conv_b / KSIZE, KSIZE)
    t_bf = pl.pallas_call(
        _table_body,
        out_shape=jax.ShapeDtypeStruct((CHAR_VOCAB, KSIZE * EMBED), jnp.bfloat16),
    )(emb_table, w2, cb)
    t_packed = lax.bitcast_convert_type(
        t_bf.reshape(CHAR_VOCAB, KSIZE * EPAIRS, 2), jnp.float32)
    t_packed = jnp.pad(t_packed, ((0, 0), (0, TROW - KSIZE * EPAIRS))
                       ).reshape(CHAR_VOCAB * TROW)

    all_ids = input.reshape(n, l).astype(jnp.int32)
    ids = jnp.swapaxes(all_ids[:N_SC].reshape(NW, WPT, L), 1, 2) * TROW
    mesh = plsc.VectorSubcoreMesh(core_axis_name="c", subcore_axis_name="s",
                                  num_cores=2, num_subcores=16)
    xc_sc = pl.kernel(
        _sc_body,
        out_type=jax.ShapeDtypeStruct((N_SC, EMBED), jnp.float32),
        mesh=mesh,
        compiler_params=pltpu.CompilerParams(needs_layout_passes=False),
        scratch_types=[
            pltpu.VMEM((CHAR_VOCAB * TROW,), jnp.float32),
            pltpu.VMEM((L, WPT), jnp.int32),
            pltpu.VMEM((2, GRP, EMBED), jnp.float32),
            pltpu.SemaphoreType.DMA,
            pltpu.SemaphoreType.DMA,
        ],
    )(ids, t_packed)

    n_tc = n - N_SC
    cblk = 512
    convw_t = jnp.transpose(conv_w, (2, 1, 0))
    xc_tc = pl.pallas_call(
        _conv_block_body,
        grid=(n_tc // cblk,),
        in_specs=[
            pl.BlockSpec((cblk, l), lambda i: (i, 0)),
            pl.BlockSpec((CHAR_VOCAB, E_CHAR), lambda i: (0, 0)),
            pl.BlockSpec((KSIZE, E_CHAR, EMBED), lambda i: (0, 0, 0)),
            pl.BlockSpec((EMBED,), lambda i: (0,)),
            pl.BlockSpec((EMBED, EMBED), lambda i: (0, 0)),
            pl.BlockSpec((EMBED,), lambda i: (0,)),
            pl.BlockSpec((EMBED, EMBED), lambda i: (0, 0)),
            pl.BlockSpec((EMBED,), lambda i: (0,)),
        ],
        out_specs=pl.BlockSpec((cblk, EMBED), lambda i: (i, 0)),
        out_shape=jax.ShapeDtypeStruct((n_tc, EMBED), jnp.float32),
    )(all_ids[N_SC:], emb_table, convw_t, conv_b,
      w_proj.T, b_proj, w_gate.T, b_gate)

    blk = 2048
    out_sc = pl.pallas_call(
        _highway_body,
        grid=(N_SC // blk,),
        in_specs=[
            pl.BlockSpec((blk, EMBED), lambda i: (i, 0)),
            pl.BlockSpec((EMBED, EMBED), lambda i: (0, 0)),
            pl.BlockSpec((EMBED,), lambda i: (0,)),
            pl.BlockSpec((EMBED, EMBED), lambda i: (0, 0)),
            pl.BlockSpec((EMBED,), lambda i: (0,)),
        ],
        out_specs=pl.BlockSpec((blk, EMBED), lambda i: (i, 0)),
        out_shape=jax.ShapeDtypeStruct((N_SC, EMBED), jnp.float32),
    )(xc_sc, w_proj.T, b_proj, w_gate.T, b_gate)
    out = jnp.concatenate([out_sc, xc_tc], axis=0)
    return out.reshape(s, b, EMBED)


# speedup vs baseline: 1.0093x; 1.0008x over previous
import jax
import jax.numpy as jnp
from jax import lax
from jax.experimental import pallas as pl
from jax.experimental.pallas import tpu as pltpu
from jax.experimental.pallas import tpu_sc as plsc

E_CHAR = 50
EMBED = 256
CHAR_VOCAB = 96
KSIZE = 5
L = 21
T_OUT = L - KSIZE + 1

NW = 32
N_SC = 10240
WPT = N_SC // NW
GRP = 16
NGRP = WPT // GRP
EPAIRS = EMBED // 2
EP_CHUNK = 2
NCHUNK = EPAIRS // EP_CHUNK
TROW = KSIZE * EPAIRS + 1


def _table_body(emb_ref, w2_ref, cb_ref, out_ref):
    t = jnp.dot(emb_ref[...], w2_ref[...], preferred_element_type=jnp.float32)
    out_ref[...] = (t + cb_ref[...][None, :]).astype(jnp.bfloat16)


def _sc_body(ids_hbm, table_hbm, out_hbm, table_v, ids_v, out_v,
             sem0, sem1):
    wid = lax.axis_index("s") * 2 + lax.axis_index("c")
    pltpu.sync_copy(table_hbm, table_v)
    pltpu.sync_copy(ids_hbm.at[wid], ids_v)
    lanes = lax.iota(jnp.int32, 16)
    sems = (sem0, sem1)

    def do_group(g, buf, sem, first):
        vb = [ids_v[j, pl.ds(g * GRP, GRP)] for j in range(L)]
        dst = out_hbm.at[pl.ds(wid * WPT + g * GRP, GRP)]

        def chunk_body(c, carry2):
            ep0 = c * EP_CHUNK
            m = [jnp.zeros((2 * GRP,), jnp.bfloat16)] * EP_CHUNK
            for t in range(T_OUT):
                acc = [None] * EP_CHUNK
                for k in range(KSIZE):
                    base = vb[t + k] + (k * EPAIRS + ep0)
                    for j in range(EP_CHUNK):
                        v = plsc.load_gather(table_v, [base + j])
                        v = plsc.bitcast(v, jnp.bfloat16)
                        acc[j] = v if k == 0 else acc[j] + v
                for j in range(EP_CHUNK):
                    m[j] = jnp.maximum(m[j], acc[j])
            for j in range(EP_CHUNK):
                lo, hi = plsc.unpack(m[j], format=plsc.PackFormat.INTERLEAVED)
                col = 2 * (ep0 + j)
                plsc.store_scatter(out_v.at[buf],
                                   [lanes, jnp.full((GRP,), 0, jnp.int32) + col], lo)
                plsc.store_scatter(out_v.at[buf],
                                   [lanes, jnp.full((GRP,), 1, jnp.int32) + col], hi)
            return carry2

        @pl.when(jnp.logical_not(first))
        def _():
            pltpu.make_async_copy(out_v.at[buf], dst, sem).wait()
        lax.fori_loop(0, NCHUNK, chunk_body, 0, unroll=False)
        pltpu.async_copy(out_v.at[buf], dst, sem)

    def group_pair(i, carry):
        do_group(2 * i, 0, sems[0], i == 0)
        do_group(2 * i + 1, 1, sems[1], i == 0)
        return carry

    lax.fori_loop(0, NGRP // 2, group_pair, 0, unroll=False)
    last = out_hbm.at[pl.ds(wid * WPT + (NGRP - 2) * GRP, GRP)]
    pltpu.make_async_copy(out_v.at[0], last, sems[0]).wait()
    pltpu.make_async_copy(out_v.at[1], last, sems[1]).wait()


def _conv_block_body(ids_ref, emb_ref, convw_ref, convb_ref,
                     wp_ref, bp_ref, wg_ref, bg_ref, out_ref):
    nb = ids_ref.shape[0]
    ids = ids_ref[...]
    iota_v = jax.lax.broadcasted_iota(jnp.int32, (nb, L, CHAR_VOCAB), 2)
    oh = (ids[:, :, None] == iota_v).astype(jnp.bfloat16)
    e = jnp.dot(oh.reshape(nb * L, CHAR_VOCAB),
                emb_ref[...].astype(jnp.bfloat16),
                preferred_element_type=jnp.float32)
    e = e.astype(jnp.bfloat16).reshape(nb, L, E_CHAR)
    acc = jnp.zeros((nb * T_OUT, EMBED), jnp.float32)
    for k in range(KSIZE):
        ek = e[:, k:k + T_OUT, :].reshape(nb * T_OUT, E_CHAR)
        acc = acc + jnp.dot(ek, convw_ref[k].astype(jnp.bfloat16),
                            preferred_element_type=jnp.float32)
    acc = acc + convb_ref[...][None, :]
    acc = jnp.maximum(acc, 0.0).reshape(nb, T_OUT, EMBED)
    xc = jnp.max(acc, axis=1)
    xcb = xc.astype(jnp.bfloat16)
    proj = jnp.maximum(
        jnp.dot(xcb, wp_ref[...].astype(jnp.bfloat16),
                preferred_element_type=jnp.float32)
        + bp_ref[...][None, :], 0.0)
    gate = jax.nn.sigmoid(
        jnp.dot(xcb, wg_ref[...].astype(jnp.bfloat16),
                preferred_element_type=jnp.float32)
        + bg_ref[...][None, :])
    out_ref[...] = gate * proj + (1.0 - gate) * xc


def _highway_body(xc_ref, wp_ref, bp_ref, wg_ref, bg_ref, out_ref):
    xc = xc_ref[...]
    xcb = xc.astype(jnp.bfloat16)
    proj = jnp.maximum(
        jnp.dot(xcb, wp_ref[...].astype(jnp.bfloat16),
                preferred_element_type=jnp.float32)
        + bp_ref[...][None, :], 0.0)
    gate = jax.nn.sigmoid(
        jnp.dot(xcb, wg_ref[...].astype(jnp.bfloat16),
                preferred_element_type=jnp.float32)
        + bg_ref[...][None, :])
    out_ref[...] = gate * proj + (1.0 - gate) * xc


def kernel(input, emb_table, conv_w, conv_b, w_proj, b_proj, w_gate, b_gate):
    s, b, l = input.shape
    n = s * b

    w2 = jnp.transpose(conv_w, (1, 2, 0)).reshape(E_CHAR, KSIZE * EMBED)
    cb = jnp.tile(conv_b / KSIZE, KSIZE)
    t_bf = pl.pallas_call(
        _table_body,
        out_shape=jax.ShapeDtypeStruct((CHAR_VOCAB, KSIZE * EMBED), jnp.bfloat16),
    )(emb_table, w2, cb)
    t_packed = lax.bitcast_convert_type(
        t_bf.reshape(CHAR_VOCAB, KSIZE * EPAIRS, 2), jnp.float32)
    t_packed = jnp.pad(t_packed, ((0, 0), (0, TROW - KSIZE * EPAIRS))
                       ).reshape(CHAR_VOCAB * TROW)

    all_ids = input.reshape(n, l).astype(jnp.int32)
    ids = jnp.swapaxes(all_ids[:N_SC].reshape(NW, WPT, L), 1, 2) * TROW
    mesh = plsc.VectorSubcoreMesh(core_axis_name="c", subcore_axis_name="s",
                                  num_cores=2, num_subcores=16)
    xc_sc = pl.kernel(
        _sc_body,
        out_type=jax.ShapeDtypeStruct((N_SC, EMBED), jnp.float32),
        mesh=mesh,
        compiler_params=pltpu.CompilerParams(needs_layout_passes=False),
        scratch_types=[
            pltpu.VMEM((CHAR_VOCAB * TROW,), jnp.float32),
            pltpu.VMEM((L, WPT), jnp.int32),
            pltpu.VMEM((2, GRP, EMBED), jnp.float32),
            pltpu.SemaphoreType.DMA,
            pltpu.SemaphoreType.DMA,
        ],
    )(ids, t_packed)

    n_tc = n - N_SC
    cblk = 512
    convw_t = jnp.transpose(conv_w, (2, 1, 0))
    xc_tc = pl.pallas_call(
        _conv_block_body,
        grid=(n_tc // cblk,),
        in_specs=[
            pl.BlockSpec((cblk, l), lambda i: (i, 0)),
            pl.BlockSpec((CHAR_VOCAB, E_CHAR), lambda i: (0, 0)),
            pl.BlockSpec((KSIZE, E_CHAR, EMBED), lambda i: (0, 0, 0)),
            pl.BlockSpec((EMBED,), lambda i: (0,)),
            pl.BlockSpec((EMBED, EMBED), lambda i: (0, 0)),
            pl.BlockSpec((EMBED,), lambda i: (0,)),
            pl.BlockSpec((EMBED, EMBED), lambda i: (0, 0)),
            pl.BlockSpec((EMBED,), lambda i: (0,)),
        ],
        out_specs=pl.BlockSpec((cblk, EMBED), lambda i: (i, 0)),
        out_shape=jax.ShapeDtypeStruct((n_tc, EMBED), jnp.float32),
    )(all_ids[N_SC:], emb_table, convw_t, conv_b,
      w_proj.T, b_proj, w_gate.T, b_gate)

    blk = 2048
    out_sc = pl.pallas_call(
        _highway_body,
        grid=(N_SC // blk,),
        in_specs=[
            pl.BlockSpec((blk, EMBED), lambda i: (i, 0)),
            pl.BlockSpec((EMBED, EMBED), lambda i: (0, 0)),
            pl.BlockSpec((EMBED,), lambda i: (0,)),
            pl.BlockSpec((EMBED, EMBED), lambda i: (0, 0)),
            pl.BlockSpec((EMBED,), lambda i: (0,)),
        ],
        out_specs=pl.BlockSpec((blk, EMBED), lambda i: (i, 0)),
        out_shape=jax.ShapeDtypeStruct((N_SC, EMBED), jnp.float32),
    )(xc_sc, w_proj.T, b_proj, w_gate.T, b_gate)
    out = jnp.concatenate([out_sc, xc_tc], axis=0)
    return out.reshape(s, b, EMBED)
